# baseline (speedup 1.0000x reference)
import jax
import jax.numpy as jnp
from jax.experimental import pallas as pl
from jax.experimental.pallas import tpu as pltpu

B, O, M = 8, 2048, 64
J = 128
OUT = 8
EPS = 1e-5


def _elu(x):
    return jnp.where(x > 0, x, jnp.exp(jnp.minimum(x, 0.0)) - 1.0)


def _norm0(x, n):
    m = jnp.mean(x, axis=0, keepdims=True)
    d = x - m
    var = jnp.sum(d * d, axis=0, keepdims=True) / (n - 1)
    return d / (jnp.sqrt(var) + EPS)


def _stats_kernel(proc_ref, adj_ref, feat_ref, wc1_ref, wc2_ref,
                  stats_ref, semb_ref):
    p = proc_ref[...]
    n = float(B * O * M)
    s = jnp.sum(p)
    mean = s / n
    var = (jnp.sum(p * p) - n * mean * mean) / (n - 1.0)
    inv = 1.0 / (jnp.sqrt(var) + EPS)
    col = jax.lax.broadcasted_iota(jnp.int32, (1, 128), 1)
    stats_ref[...] = jnp.where(col == 0, mean, jnp.where(col == 1, inv, 0.0))

    adjsum = jnp.sum(adj_ref[...].astype(jnp.float32), axis=(1, 2),
                     keepdims=True).reshape(B, 1)
    yi = jnp.sum(feat_ref[...][:, :, 0:1], axis=(1, 2),
                 keepdims=True).reshape(B, 1)
    y = adjsum - yi
    q = jnp.float32(O) - yi
    d1 = float(int(J * M * (M + 1) / 2))
    d2 = float(int(J * M * 1.2))
    si = jnp.concatenate([y / d1, q / d2], axis=1)
    h = jnp.maximum(jnp.dot(si, wc1_ref[...],
                            preferred_element_type=jnp.float32), 0.0)
    semb_ref[...] = jnp.dot(h, wc2_ref[...],
                            preferred_element_type=jnp.float32)


def _gat(xop, xma, pn, adjf, deg, wop, wma, aop, ama, wp, ws, wm, wo):
    f32 = jnp.float32
    v = jax.lax.dot_general(wop, aop, (((1,), (1,)), ((), ())),
                            preferred_element_type=f32)
    wcat = jnp.concatenate([wo, ws, wop, v], axis=1)
    ecat = jnp.dot(xop, wcat, preferred_element_type=f32)
    e = ecat[:, 0:8]
    s0 = ecat[:, 8:16]
    e_op = ecat[:, 16:24]
    lo = ecat[:, 24:25]
    e_ma = jnp.dot(xma, wma, preferred_element_type=f32)
    lm = jax.lax.dot_general(ama, e_ma, (((1,), (1,)), ((), ())),
                             preferred_element_type=f32)
    logits = lo + lm + wp * pn
    logits = jnp.where(logits > 0, logits, 0.2 * logits)
    logits = jnp.where(adjf > 0, logits, -1e9)
    mx = jnp.max(logits, axis=0, keepdims=True)
    ex = jnp.exp(logits - mx)
    attn = ex / jnp.sum(ex, axis=0, keepdims=True)
    agg = jax.lax.dot_general(attn, e_op, (((0,), (0,)), ((), ())),
                              preferred_element_type=f32)
    h_ma = _elu(agg + e_ma)
    mm = jnp.dot(h_ma, wm, preferred_element_type=f32)
    msg_m = jnp.dot(adjf, mm, preferred_element_type=f32) / deg
    s = s0 + msg_m
    return h_ma, s, e


def _mega_kernel(adjoo_ref, fo_ref, fm_ref, proc_ref, adjma_ref, stats_ref,
                 wop1_ref, wma1_ref, aop1_ref, ama1_ref, wp1_ref,
                 wop2_ref, wma2_ref, aop2_ref, ama2_ref, wp2_ref,
                 ws1_ref, wm1_ref, wo1_ref, ws2_ref, wm2_ref, wo2_ref,
                 hop_ref, hma_ref):
    f32 = jnp.float32
    bf16 = jnp.bfloat16
    abf = adjoo_ref[0].astype(bf16)

    pn = (proc_ref[0] - stats_ref[0, 0]) * stats_ref[0, 1]
    adjf = adjma_ref[0].astype(f32)
    deg_m = jnp.maximum(jnp.sum(adjf, axis=1, keepdims=True), 1.0)
    ones_col = jnp.ones((O, 1), dtype=bf16)

    xop = _norm0(fo_ref[0], O)
    xma = _norm0(fm_ref[0], M)
    h_ma, s, e = _gat(xop, xma, pn, adjf, deg_m,
                      wop1_ref[...], wma1_ref[...], aop1_ref[...],
                      ama1_ref[...], wp1_ref[0, 0],
                      ws1_ref[...], wm1_ref[...], wo1_ref[...])
    eaug = jnp.concatenate([e.astype(bf16), ones_col], axis=1)
    msg = jnp.dot(abf, eaug, preferred_element_type=f32)
    deg = jnp.maximum(msg[:, 8:9], 1.0)
    h_op = _elu(s + msg[:, :8] / deg)

    h_ma, s, e = _gat(h_op, h_ma, pn, adjf, deg_m,
                      wop2_ref[...], wma2_ref[...], aop2_ref[...],
                      ama2_ref[...], wp2_ref[0, 0],
                      ws2_ref[...], wm2_ref[...], wo2_ref[...])
    eaug = jnp.concatenate([e.astype(bf16), ones_col], axis=1)
    msg = jnp.dot(abf, eaug, preferred_element_type=f32)
    hop_ref[0] = _elu(s + msg[:, :8] / deg)
    hma_ref[0] = h_ma


def _head_kernel(ho_ref, hm_ref, semb_ref, wp1_ref, wp2_ref,
                 vc1_ref, bc1_ref, vc2_ref, bc2_ref, vc3_ref, bc3_ref,
                 out_ref):
    f32 = jnp.float32
    op = jnp.mean(ho_ref[...], axis=1)
    mp = jnp.mean(hm_ref[...], axis=1)
    se = semb_ref[...]

    def proj(x):
        h = jnp.maximum(jnp.dot(x, wp1_ref[...],
                                preferred_element_type=f32), 0.0)
        return jnp.dot(h, wp2_ref[...], preferred_element_type=f32)

    op = op + proj(op + se)
    mp = mp + proj(mp + se)
    hp = jnp.concatenate([op, mp], axis=1)
    h = jnp.tanh(jnp.dot(hp, vc1_ref[...], preferred_element_type=f32)
                 + bc1_ref[...])
    h = jnp.tanh(jnp.dot(h, vc2_ref[...], preferred_element_type=f32)
                 + bc2_ref[...])
    v = jnp.dot(h, vc3_ref[...], preferred_element_type=f32) + bc3_ref[...]
    out_ref[...] = jnp.broadcast_to(v, (B, 128))


def kernel(feat_opes, feat_mas, proc_time, ope_ma_adj, ope_ope_adj,
           W_op1, W_ma1, a_op1, a_ma1, w_p1,
           W_op2, W_ma2, a_op2, a_ma2, w_p2,
           Ws1, Wm1, Wo1, Ws2, Wm2, Wo2,
           Wc1, Wc2, Wp1, Wp2, Vc1, bc1, Vc2, bc2, Vc3, bc3):
    f32 = jnp.float32
    a_op1 = a_op1.reshape(1, OUT)
    a_ma1 = a_ma1.reshape(1, OUT)
    a_op2 = a_op2.reshape(1, OUT)
    a_ma2 = a_ma2.reshape(1, OUT)
    w_p1 = w_p1.reshape(1, 1)
    w_p2 = w_p2.reshape(1, 1)
    bc1 = bc1.reshape(1, 64)
    bc2 = bc2.reshape(1, 64)
    bc3 = bc3.reshape(1, 1)

    stats, semb = pl.pallas_call(
        _stats_kernel,
        out_shape=[jax.ShapeDtypeStruct((1, 128), f32),
                   jax.ShapeDtypeStruct((B, OUT), f32)],
    )(proc_time, ope_ma_adj, feat_opes, Wc1, Wc2)

    bspec = lambda shp: pl.BlockSpec(shp, lambda b: (b, 0, 0))
    full2 = lambda arr: pl.BlockSpec(arr.shape, lambda b: (0, 0))
    h_op2, h_ma2 = pl.pallas_call(
        _mega_kernel,
        grid=(B,),
        in_specs=[bspec((1, O, O)),
                  bspec((1, O, 12)), bspec((1, M, 6)),
                  bspec((1, O, M)), bspec((1, O, M)),
                  full2(stats),
                  full2(W_op1), full2(W_ma1), full2(a_op1), full2(a_ma1),
                  full2(w_p1),
                  full2(W_op2), full2(W_ma2), full2(a_op2), full2(a_ma2),
                  full2(w_p2),
                  full2(Ws1), full2(Wm1), full2(Wo1),
                  full2(Ws2), full2(Wm2), full2(Wo2)],
        out_specs=[bspec((1, O, OUT)), bspec((1, M, OUT))],
        out_shape=[jax.ShapeDtypeStruct((B, O, OUT), f32),
                   jax.ShapeDtypeStruct((B, M, OUT), f32)],
        compiler_params=pltpu.CompilerParams(
            dimension_semantics=("arbitrary",),
            vmem_limit_bytes=112 * 1024 * 1024),
    )(ope_ope_adj, feat_opes, feat_mas, proc_time, ope_ma_adj, stats,
      W_op1, W_ma1, a_op1, a_ma1, w_p1,
      W_op2, W_ma2, a_op2, a_ma2, w_p2,
      Ws1, Wm1, Wo1, Ws2, Wm2, Wo2)

    head = pl.pallas_call(
        _head_kernel,
        out_shape=jax.ShapeDtypeStruct((B, 128), f32),
    )(h_op2, h_ma2, semb, Wp1, Wp2, Vc1, bc1, Vc2, bc2, Vc3, bc3)
    value = head[:, :1]

    return jnp.concatenate([h_op2.reshape(B, O * OUT),
                            h_ma2.reshape(B, M * OUT), value], axis=1)


# speedup vs baseline: 1.0059x; 1.0059x over previous
import jax
import jax.numpy as jnp
from jax.experimental import pallas as pl
from jax.experimental.pallas import tpu as pltpu

B, O, M = 8, 2048, 64
J = 128
OUT = 8
EPS = 1e-5


def _elu(x):
    return jnp.where(x > 0, x, jnp.exp(jnp.minimum(x, 0.0)) - 1.0)


def _norm0(x, n):
    m = jnp.mean(x, axis=0, keepdims=True)
    d = x - m
    var = jnp.sum(d * d, axis=0, keepdims=True) / (n - 1)
    return d / (jnp.sqrt(var) + EPS)


def _stats_kernel(proc_ref, adj_ref, feat_ref, wc1_ref, wc2_ref,
                  stats_ref, semb_ref):
    p = proc_ref[...]
    n = float(B * O * M)
    s = jnp.sum(p)
    mean = s / n
    var = (jnp.sum(p * p) - n * mean * mean) / (n - 1.0)
    inv = 1.0 / (jnp.sqrt(var) + EPS)
    col = jax.lax.broadcasted_iota(jnp.int32, (1, 128), 1)
    stats_ref[...] = jnp.where(col == 0, mean, jnp.where(col == 1, inv, 0.0))

    adjsum = jnp.sum(adj_ref[...].astype(jnp.float32), axis=(1, 2),
                     keepdims=True).reshape(B, 1)
    yi = jnp.sum(feat_ref[...][:, :, 0:1], axis=(1, 2),
                 keepdims=True).reshape(B, 1)
    y = adjsum - yi
    q = jnp.float32(O) - yi
    d1 = float(int(J * M * (M + 1) / 2))
    d2 = float(int(J * M * 1.2))
    si = jnp.concatenate([y / d1, q / d2], axis=1)
    h = jnp.maximum(jnp.dot(si, wc1_ref[...],
                            preferred_element_type=jnp.float32), 0.0)
    semb_ref[...] = jnp.dot(h, wc2_ref[...],
                            preferred_element_type=jnp.float32)


def _gat(xop, xma, pnT, adjfT, deg_col, wop, wma, aop, ama, wp, ws, wm, wo):
    f32 = jnp.float32
    v = jax.lax.dot_general(wop, aop, (((1,), (1,)), ((), ())),
                            preferred_element_type=f32)
    wcat = jnp.concatenate([wo, ws, wop, v], axis=1)
    ecat = jnp.dot(xop, wcat, preferred_element_type=f32)
    e = ecat[:, 0:8]
    s0 = ecat[:, 8:16]
    e_op = ecat[:, 16:24]
    loT = jax.lax.transpose(ecat[:, 24:25], (1, 0))
    e_ma = jnp.dot(xma, wma, preferred_element_type=f32)
    lm_col = jax.lax.dot_general(e_ma, ama, (((1,), (1,)), ((), ())),
                                 preferred_element_type=f32)
    logits = loT + lm_col + wp * pnT
    logits = jnp.where(logits > 0, logits, 0.2 * logits)
    logits = jnp.where(adjfT > 0, logits, -1e9)
    mx = jnp.max(logits, axis=1, keepdims=True)
    ex = jnp.exp(logits - mx)
    attnT = ex / jnp.sum(ex, axis=1, keepdims=True)
    agg = jnp.dot(attnT, e_op, preferred_element_type=f32)
    h_ma = _elu(agg + e_ma)
    mm = jnp.dot(h_ma, wm, preferred_element_type=f32)
    msg_m = jax.lax.dot_general(adjfT, mm, (((0,), (0,)), ((), ())),
                                preferred_element_type=f32)
    s = s0 + msg_m / deg_col
    return h_ma, s, e


def _mega_kernel(adjoo_ref, fo_ref, fm_ref, proc_ref, adjma_ref, stats_ref,
                 wop1_ref, wma1_ref, aop1_ref, ama1_ref, wp1_ref,
                 wop2_ref, wma2_ref, aop2_ref, ama2_ref, wp2_ref,
                 ws1_ref, wm1_ref, wo1_ref, ws2_ref, wm2_ref, wo2_ref,
                 hop_ref, hma_ref):
    f32 = jnp.float32
    bf16 = jnp.bfloat16
    abf = adjoo_ref[0].astype(bf16)

    pnT = (proc_ref[0] - stats_ref[0, 0]) * stats_ref[0, 1]
    adjfT = adjma_ref[0].astype(f32)
    deg_mT = jnp.maximum(jnp.sum(adjfT, axis=0, keepdims=True), 1.0)
    deg_col = jax.lax.transpose(deg_mT, (1, 0))
    ones_col = jnp.ones((O, 1), dtype=bf16)

    xop = _norm0(fo_ref[0], O)
    xma = _norm0(fm_ref[0], M)
    h_ma, s, e = _gat(xop, xma, pnT, adjfT, deg_col,
                      wop1_ref[...], wma1_ref[...], aop1_ref[...],
                      ama1_ref[...], wp1_ref[0, 0],
                      ws1_ref[...], wm1_ref[...], wo1_ref[...])
    eaug = jnp.concatenate([e.astype(bf16), ones_col], axis=1)
    msg = jnp.dot(abf, eaug, preferred_element_type=f32)
    deg = jnp.maximum(msg[:, 8:9], 1.0)
    h_op = _elu(s + msg[:, :8] / deg)

    h_ma, s, e = _gat(h_op, h_ma, pnT, adjfT, deg_col,
                      wop2_ref[...], wma2_ref[...], aop2_ref[...],
                      ama2_ref[...], wp2_ref[0, 0],
                      ws2_ref[...], wm2_ref[...], wo2_ref[...])
    eaug = jnp.concatenate([e.astype(bf16), ones_col], axis=1)
    msg = jnp.dot(abf, eaug, preferred_element_type=f32)
    hop_ref[0] = _elu(s + msg[:, :8] / deg)
    hma_ref[0] = h_ma


def _head_kernel(ho_ref, hm_ref, semb_ref, wp1_ref, wp2_ref,
                 vc1_ref, bc1_ref, vc2_ref, bc2_ref, vc3_ref, bc3_ref,
                 out_ref):
    f32 = jnp.float32
    op = jnp.mean(ho_ref[...], axis=1)
    mp = jnp.mean(hm_ref[...], axis=1)
    se = semb_ref[...]

    def proj(x):
        h = jnp.maximum(jnp.dot(x, wp1_ref[...],
                                preferred_element_type=f32), 0.0)
        return jnp.dot(h, wp2_ref[...], preferred_element_type=f32)

    op = op + proj(op + se)
    mp = mp + proj(mp + se)
    hp = jnp.concatenate([op, mp], axis=1)
    h = jnp.tanh(jnp.dot(hp, vc1_ref[...], preferred_element_type=f32)
                 + bc1_ref[...])
    h = jnp.tanh(jnp.dot(h, vc2_ref[...], preferred_element_type=f32)
                 + bc2_ref[...])
    v = jnp.dot(h, vc3_ref[...], preferred_element_type=f32) + bc3_ref[...]
    out_ref[...] = jnp.broadcast_to(v, (B, 128))


def kernel(feat_opes, feat_mas, proc_time, ope_ma_adj, ope_ope_adj,
           W_op1, W_ma1, a_op1, a_ma1, w_p1,
           W_op2, W_ma2, a_op2, a_ma2, w_p2,
           Ws1, Wm1, Wo1, Ws2, Wm2, Wo2,
           Wc1, Wc2, Wp1, Wp2, Vc1, bc1, Vc2, bc2, Vc3, bc3):
    f32 = jnp.float32
    a_op1 = a_op1.reshape(1, OUT)
    a_ma1 = a_ma1.reshape(1, OUT)
    a_op2 = a_op2.reshape(1, OUT)
    a_ma2 = a_ma2.reshape(1, OUT)
    w_p1 = w_p1.reshape(1, 1)
    w_p2 = w_p2.reshape(1, 1)
    bc1 = bc1.reshape(1, 64)
    bc2 = bc2.reshape(1, 64)
    bc3 = bc3.reshape(1, 1)

    stats, semb = pl.pallas_call(
        _stats_kernel,
        out_shape=[jax.ShapeDtypeStruct((1, 128), f32),
                   jax.ShapeDtypeStruct((B, OUT), f32)],
    )(proc_time, ope_ma_adj, feat_opes, Wc1, Wc2)

    proc_t = jnp.swapaxes(proc_time, 1, 2)
    adjma_t = jnp.swapaxes(ope_ma_adj, 1, 2)
    bspec = lambda shp: pl.BlockSpec(shp, lambda b: (b, 0, 0))
    full2 = lambda arr: pl.BlockSpec(arr.shape, lambda b: (0, 0))
    h_op2, h_ma2 = pl.pallas_call(
        _mega_kernel,
        grid=(B,),
        in_specs=[bspec((1, O, O)),
                  bspec((1, O, 12)), bspec((1, M, 6)),
                  bspec((1, M, O)), bspec((1, M, O)),
                  full2(stats),
                  full2(W_op1), full2(W_ma1), full2(a_op1), full2(a_ma1),
                  full2(w_p1),
                  full2(W_op2), full2(W_ma2), full2(a_op2), full2(a_ma2),
                  full2(w_p2),
                  full2(Ws1), full2(Wm1), full2(Wo1),
                  full2(Ws2), full2(Wm2), full2(Wo2)],
        out_specs=[bspec((1, O, OUT)), bspec((1, M, OUT))],
        out_shape=[jax.ShapeDtypeStruct((B, O, OUT), f32),
                   jax.ShapeDtypeStruct((B, M, OUT), f32)],
        compiler_params=pltpu.CompilerParams(
            dimension_semantics=("arbitrary",),
            vmem_limit_bytes=112 * 1024 * 1024),
    )(ope_ope_adj, feat_opes, feat_mas, proc_t, adjma_t, stats,
      W_op1, W_ma1, a_op1, a_ma1, w_p1,
      W_op2, W_ma2, a_op2, a_ma2, w_p2,
      Ws1, Wm1, Wo1, Ws2, Wm2, Wo2)

    head = pl.pallas_call(
        _head_kernel,
        out_shape=jax.ShapeDtypeStruct((B, 128), f32),
    )(h_op2, h_ma2, semb, Wp1, Wp2, Vc1, bc1, Vc2, bc2, Vc3, bc3)
    value = head[:, :1]

    return jnp.concatenate([h_op2.reshape(B, O * OUT),
                            h_ma2.reshape(B, M * OUT), value], axis=1)


# speedup vs baseline: 1.1249x; 1.1184x over previous
import jax
import jax.numpy as jnp
from jax.experimental import pallas as pl
from jax.experimental.pallas import tpu as pltpu

B, O, M = 8, 2048, 64
J = 128
OUT = 8
EPS = 1e-5


def _elu(x):
    return jnp.where(x > 0, x, jnp.exp(jnp.minimum(x, 0.0)) - 1.0)


def _norm0(x, n):
    m = jnp.mean(x, axis=0, keepdims=True)
    d = x - m
    var = jnp.sum(d * d, axis=0, keepdims=True) / (n - 1)
    return d / (jnp.sqrt(var) + EPS)


def _stats_kernel(proc_ref, adj_ref, feat_ref, wc1_ref, wc2_ref,
                  stats_ref, semb_ref):
    p = proc_ref[...]
    n = float(B * O * M)
    s = jnp.sum(p)
    mean = s / n
    var = (jnp.sum(p * p) - n * mean * mean) / (n - 1.0)
    inv = 1.0 / (jnp.sqrt(var) + EPS)
    col = jax.lax.broadcasted_iota(jnp.int32, (1, 128), 1)
    stats_ref[...] = jnp.where(col == 0, mean, jnp.where(col == 1, inv, 0.0))

    adjsum = jnp.sum(adj_ref[...].astype(jnp.float32), axis=(1, 2),
                     keepdims=True).reshape(B, 1)
    yi = jnp.sum(feat_ref[...][:, :, 0:1], axis=(1, 2),
                 keepdims=True).reshape(B, 1)
    y = adjsum - yi
    q = jnp.float32(O) - yi
    d1 = float(int(J * M * (M + 1) / 2))
    d2 = float(int(J * M * 1.2))
    si = jnp.concatenate([y / d1, q / d2], axis=1)
    h = jnp.maximum(jnp.dot(si, wc1_ref[...],
                            preferred_element_type=jnp.float32), 0.0)
    semb_ref[...] = jnp.dot(h, wc2_ref[...],
                            preferred_element_type=jnp.float32)


def _gat(xop, xma, pnT, adjfT, deg_col, wop, wma, aop, ama, wp, ws, wm, wo):
    f32 = jnp.float32
    v = jax.lax.dot_general(wop, aop, (((1,), (1,)), ((), ())),
                            preferred_element_type=f32)
    wcat = jnp.concatenate([wo, ws, wop, v], axis=1)
    ecat = jnp.dot(xop, wcat, preferred_element_type=f32)
    e = ecat[:, 0:8]
    s0 = ecat[:, 8:16]
    e_op = ecat[:, 16:24]
    loT = jax.lax.transpose(ecat[:, 24:25], (1, 0))
    e_ma = jnp.dot(xma, wma, preferred_element_type=f32)
    lm_col = jax.lax.dot_general(e_ma, ama, (((1,), (1,)), ((), ())),
                                 preferred_element_type=f32)
    logits = loT + lm_col + wp * pnT
    logits = jnp.where(logits > 0, logits, 0.2 * logits)
    logits = jnp.where(adjfT > 0, logits, -1e9)
    mx = jnp.max(logits, axis=1, keepdims=True)
    ex = jnp.exp(logits - mx)
    attnT = ex / jnp.sum(ex, axis=1, keepdims=True)
    agg = jnp.dot(attnT, e_op, preferred_element_type=f32)
    h_ma = _elu(agg + e_ma)
    mm = jnp.dot(h_ma, wm, preferred_element_type=f32)
    msg_m = jax.lax.dot_general(adjfT, mm, (((0,), (0,)), ((), ())),
                                preferred_element_type=f32)
    s = s0 + msg_m / deg_col
    return h_ma, s, e


def _mega_kernel(adjoo_ref, fo_ref, fm_ref, proc_ref, adjma_ref,
                 wop1_ref, wma1_ref, aop1_ref, ama1_ref, wp1_ref,
                 wop2_ref, wma2_ref, aop2_ref, ama2_ref, wp2_ref,
                 ws1_ref, wm1_ref, wo1_ref, ws2_ref, wm2_ref, wo2_ref,
                 hop_ref, hma_ref, misc_ref, smem_ref):
    f32 = jnp.float32
    bf16 = jnp.bfloat16
    b = pl.program_id(0)

    @pl.when(b == 0)
    def _():
        p = proc_ref[...]
        n = float(B * O * M)
        mean = jnp.sum(p) / n
        var = (jnp.sum(p * p) - n * mean * mean) / (n - 1.0)
        smem_ref[0] = mean
        smem_ref[1] = 1.0 / (jnp.sqrt(var) + EPS)

    abf = adjoo_ref[0].astype(bf16)
    pnT = (proc_ref[b] - smem_ref[0]) * smem_ref[1]
    adjfT = adjma_ref[0].astype(f32)
    deg_mT = jnp.maximum(jnp.sum(adjfT, axis=0, keepdims=True), 1.0)
    deg_col = jax.lax.transpose(deg_mT, (1, 0))
    ones_col = jnp.ones((O, 1), dtype=bf16)

    xop = _norm0(fo_ref[0], O)
    xma = _norm0(fm_ref[0], M)
    h_ma, s, e = _gat(xop, xma, pnT, adjfT, deg_col,
                      wop1_ref[...], wma1_ref[...], aop1_ref[...],
                      ama1_ref[...], wp1_ref[0, 0],
                      ws1_ref[...], wm1_ref[...], wo1_ref[...])
    eaug = jnp.concatenate([e.astype(bf16), ones_col], axis=1)
    msg = jnp.dot(abf, eaug, preferred_element_type=f32)
    deg = jnp.maximum(msg[:, 8:9], 1.0)
    h_op = _elu(s + msg[:, :8] / deg)

    h_ma, s, e = _gat(h_op, h_ma, pnT, adjfT, deg_col,
                      wop2_ref[...], wma2_ref[...], aop2_ref[...],
                      ama2_ref[...], wp2_ref[0, 0],
                      ws2_ref[...], wm2_ref[...], wo2_ref[...])
    eaug = jnp.concatenate([e.astype(bf16), ones_col], axis=1)
    msg = jnp.dot(abf, eaug, preferred_element_type=f32)
    hop_ref[0] = _elu(s + msg[:, :8] / deg)
    hma_ref[0] = h_ma
    adjsum = jnp.sum(adjfT)
    yi = jnp.sum(fo_ref[0][:, 0:1])
    col = jax.lax.broadcasted_iota(jnp.int32, (1, 128), 1)
    row = jnp.where(col == 0, adjsum, jnp.where(col == 1, yi, 0.0))
    misc_ref[0] = jnp.broadcast_to(row, (8, 128))


def _head_kernel(ho_ref, hm_ref, misc_ref, wc1_ref, wc2_ref, wp1_ref,
                 wp2_ref, vc1_ref, bc1_ref, vc2_ref, bc2_ref, vc3_ref,
                 bc3_ref, out_ref):
    f32 = jnp.float32
    op = jnp.mean(ho_ref[...], axis=1)
    mp = jnp.mean(hm_ref[...], axis=1)
    m = misc_ref[...]
    adjsum = m[:, 0, 0:1]
    yi = m[:, 0, 1:2]
    y = adjsum - yi
    q = jnp.float32(O) - yi
    d1 = float(int(J * M * (M + 1) / 2))
    d2 = float(int(J * M * 1.2))
    si = jnp.concatenate([y / d1, q / d2], axis=1)
    se = jnp.maximum(jnp.dot(si, wc1_ref[...],
                             preferred_element_type=f32), 0.0)
    se = jnp.dot(se, wc2_ref[...], preferred_element_type=f32)

    def proj(x):
        h = jnp.maximum(jnp.dot(x, wp1_ref[...],
                                preferred_element_type=f32), 0.0)
        return jnp.dot(h, wp2_ref[...], preferred_element_type=f32)

    op = op + proj(op + se)
    mp = mp + proj(mp + se)
    hp = jnp.concatenate([op, mp], axis=1)
    h = jnp.tanh(jnp.dot(hp, vc1_ref[...], preferred_element_type=f32)
                 + bc1_ref[...])
    h = jnp.tanh(jnp.dot(h, vc2_ref[...], preferred_element_type=f32)
                 + bc2_ref[...])
    v = jnp.dot(h, vc3_ref[...], preferred_element_type=f32) + bc3_ref[...]
    out_ref[...] = jnp.broadcast_to(v, (B, 128))


def kernel(feat_opes, feat_mas, proc_time, ope_ma_adj, ope_ope_adj,
           W_op1, W_ma1, a_op1, a_ma1, w_p1,
           W_op2, W_ma2, a_op2, a_ma2, w_p2,
           Ws1, Wm1, Wo1, Ws2, Wm2, Wo2,
           Wc1, Wc2, Wp1, Wp2, Vc1, bc1, Vc2, bc2, Vc3, bc3):
    f32 = jnp.float32
    a_op1 = a_op1.reshape(1, OUT)
    a_ma1 = a_ma1.reshape(1, OUT)
    a_op2 = a_op2.reshape(1, OUT)
    a_ma2 = a_ma2.reshape(1, OUT)
    w_p1 = w_p1.reshape(1, 1)
    w_p2 = w_p2.reshape(1, 1)
    bc1 = bc1.reshape(1, 64)
    bc2 = bc2.reshape(1, 64)
    bc3 = bc3.reshape(1, 1)

    proc_t = jnp.swapaxes(proc_time, 1, 2)
    adjma_t = jnp.swapaxes(ope_ma_adj, 1, 2)
    bspec = lambda shp: pl.BlockSpec(shp, lambda b: (b, 0, 0))
    full2 = lambda arr: pl.BlockSpec(arr.shape, lambda b: (0, 0))
    full3 = lambda arr: pl.BlockSpec(arr.shape, lambda b: (0, 0, 0))
    h_op2, h_ma2, misc = pl.pallas_call(
        _mega_kernel,
        grid=(B,),
        in_specs=[bspec((1, O, O)),
                  bspec((1, O, 12)), bspec((1, M, 6)),
                  full3(proc_t), bspec((1, M, O)),
                  full2(W_op1), full2(W_ma1), full2(a_op1), full2(a_ma1),
                  full2(w_p1),
                  full2(W_op2), full2(W_ma2), full2(a_op2), full2(a_ma2),
                  full2(w_p2),
                  full2(Ws1), full2(Wm1), full2(Wo1),
                  full2(Ws2), full2(Wm2), full2(Wo2)],
        out_specs=[bspec((1, O, OUT)), bspec((1, M, OUT)),
                   bspec((1, 8, 128))],
        out_shape=[jax.ShapeDtypeStruct((B, O, OUT), f32),
                   jax.ShapeDtypeStruct((B, M, OUT), f32),
                   jax.ShapeDtypeStruct((B, 8, 128), f32)],
        scratch_shapes=[pltpu.SMEM((2,), f32)],
        compiler_params=pltpu.CompilerParams(
            dimension_semantics=("arbitrary",),
            vmem_limit_bytes=112 * 1024 * 1024),
    )(ope_ope_adj, feat_opes, feat_mas, proc_t, adjma_t,
      W_op1, W_ma1, a_op1, a_ma1, w_p1,
      W_op2, W_ma2, a_op2, a_ma2, w_p2,
      Ws1, Wm1, Wo1, Ws2, Wm2, Wo2)

    head = pl.pallas_call(
        _head_kernel,
        out_shape=jax.ShapeDtypeStruct((B, 128), f32),
    )(h_op2, h_ma2, misc, Wc1, Wc2, Wp1, Wp2, Vc1, bc1, Vc2, bc2, Vc3, bc3)
    value = head[:, :1]

    return jnp.concatenate([h_op2.reshape(B, O * OUT),
                            h_ma2.reshape(B, M * OUT), value], axis=1)


# speedup vs baseline: 1.1256x; 1.0006x over previous
import jax
import jax.numpy as jnp
from jax.experimental import pallas as pl
from jax.experimental.pallas import tpu as pltpu

B, O, M = 8, 2048, 64
J = 128
OUT = 8
EPS = 1e-5


def _elu(x):
    return jnp.where(x > 0, x, jnp.exp(jnp.minimum(x, 0.0)) - 1.0)


def _norm0(x, n):
    m = jnp.mean(x, axis=0, keepdims=True)
    d = x - m
    var = jnp.sum(d * d, axis=0, keepdims=True) / (n - 1)
    return d / (jnp.sqrt(var) + EPS)


def _gat(xop, xma, pnT, adjfT, deg_col, wop, wma, aop, ama, wp, ws, wm, wo):
    f32 = jnp.float32
    v = jax.lax.dot_general(wop, aop, (((1,), (1,)), ((), ())),
                            preferred_element_type=f32)
    wcat = jnp.concatenate([wo, ws, wop, v], axis=1)
    ecat = jnp.dot(xop, wcat, preferred_element_type=f32)
    e = ecat[:, 0:8]
    s0 = ecat[:, 8:16]
    e_op = ecat[:, 16:24]
    loT = jax.lax.transpose(ecat[:, 24:25], (1, 0))
    e_ma = jnp.dot(xma, wma, preferred_element_type=f32)
    lm_col = jax.lax.dot_general(e_ma, ama, (((1,), (1,)), ((), ())),
                                 preferred_element_type=f32)
    logits = loT + lm_col + wp * pnT
    logits = jnp.where(logits > 0, logits, 0.2 * logits)
    logits = jnp.where(adjfT > 0, logits, -1e9)
    mx = jnp.max(logits, axis=1, keepdims=True)
    ex = jnp.exp(logits - mx)
    attnT = ex / jnp.sum(ex, axis=1, keepdims=True)
    agg = jnp.dot(attnT, e_op, preferred_element_type=f32)
    h_ma = _elu(agg + e_ma)
    mm = jnp.dot(h_ma, wm, preferred_element_type=f32)
    msg_m = jax.lax.dot_general(adjfT, mm, (((0,), (0,)), ((), ())),
                                preferred_element_type=f32)
    s = s0 + msg_m / deg_col
    return h_ma, s, e


def _mega_kernel(adjoo_ref, fo_ref, fm_ref, proc_ref, adjma_ref,
                 wop1_ref, wma1_ref, aop1_ref, ama1_ref, wp1_ref,
                 wop2_ref, wma2_ref, aop2_ref, ama2_ref, wp2_ref,
                 ws1_ref, wm1_ref, wo1_ref, ws2_ref, wm2_ref, wo2_ref,
                 hop_ref, hma_ref, misc_ref, smem_ref):
    f32 = jnp.float32
    bf16 = jnp.bfloat16
    b = pl.program_id(0)

    @pl.when(b == 0)
    def _():
        p = proc_ref[...]
        n = float(B * O * M)
        mean = jnp.sum(p) / n
        var = (jnp.sum(p * p) - n * mean * mean) / (n - 1.0)
        smem_ref[0] = mean
        smem_ref[1] = 1.0 / (jnp.sqrt(var) + EPS)

    abf = adjoo_ref[0].astype(bf16)
    pnT = (proc_ref[b] - smem_ref[0]) * smem_ref[1]
    adjfT = adjma_ref[0].astype(f32)
    deg_mT = jnp.maximum(jnp.sum(adjfT, axis=0, keepdims=True), 1.0)
    deg_col = jax.lax.transpose(deg_mT, (1, 0))
    ones_col = jnp.ones((O, 1), dtype=bf16)

    xop = _norm0(fo_ref[0], O)
    xma = _norm0(fm_ref[0], M)
    h_ma, s, e = _gat(xop, xma, pnT, adjfT, deg_col,
                      wop1_ref[...], wma1_ref[...], aop1_ref[...],
                      ama1_ref[...], wp1_ref[0, 0],
                      ws1_ref[...], wm1_ref[...], wo1_ref[...])
    eaug = jnp.concatenate([e.astype(bf16), ones_col], axis=1)
    msg = jnp.dot(abf, eaug, preferred_element_type=f32)
    deg = jnp.maximum(msg[:, 8:9], 1.0)
    h_op = _elu(s + msg[:, :8] / deg)

    h_ma, s, e = _gat(h_op, h_ma, pnT, adjfT, deg_col,
                      wop2_ref[...], wma2_ref[...], aop2_ref[...],
                      ama2_ref[...], wp2_ref[0, 0],
                      ws2_ref[...], wm2_ref[...], wo2_ref[...])
    eaug = jnp.concatenate([e.astype(bf16), ones_col], axis=1)
    msg = jnp.dot(abf, eaug, preferred_element_type=f32)
    hop_ref[0] = _elu(s + msg[:, :8] / deg)
    hma_ref[0] = h_ma
    adjsum = jnp.sum(adjfT)
    yi = jnp.sum(fo_ref[0][:, 0:1])
    col = jax.lax.broadcasted_iota(jnp.int32, (1, 128), 1)
    row = jnp.where(col == 0, adjsum, jnp.where(col == 1, yi, 0.0))
    misc_ref[0] = jnp.broadcast_to(row, (8, 128))


def _head_kernel(ho_ref, hm_ref, misc_ref, wc1_ref, wc2_ref, wp1_ref,
                 wp2_ref, vc1_ref, bc1_ref, vc2_ref, bc2_ref, vc3_ref,
                 bc3_ref, out_ref):
    f32 = jnp.float32
    op = jnp.mean(ho_ref[...], axis=1)
    mp = jnp.mean(hm_ref[...], axis=1)
    m = misc_ref[...]
    adjsum = m[:, 0, 0:1]
    yi = m[:, 0, 1:2]
    y = adjsum - yi
    q = jnp.float32(O) - yi
    d1 = float(int(J * M * (M + 1) / 2))
    d2 = float(int(J * M * 1.2))
    si = jnp.concatenate([y / d1, q / d2], axis=1)
    se = jnp.maximum(jnp.dot(si, wc1_ref[...],
                             preferred_element_type=f32), 0.0)
    se = jnp.dot(se, wc2_ref[...], preferred_element_type=f32)

    def proj(x):
        h = jnp.maximum(jnp.dot(x, wp1_ref[...],
                                preferred_element_type=f32), 0.0)
        return jnp.dot(h, wp2_ref[...], preferred_element_type=f32)

    op = op + proj(op + se)
    mp = mp + proj(mp + se)
    hp = jnp.concatenate([op, mp], axis=1)
    h = jnp.tanh(jnp.dot(hp, vc1_ref[...], preferred_element_type=f32)
                 + bc1_ref[...])
    h = jnp.tanh(jnp.dot(h, vc2_ref[...], preferred_element_type=f32)
                 + bc2_ref[...])
    v = jnp.dot(h, vc3_ref[...], preferred_element_type=f32) + bc3_ref[...]
    out_ref[...] = jnp.broadcast_to(v, (B, 128))


def kernel(feat_opes, feat_mas, proc_time, ope_ma_adj, ope_ope_adj,
           W_op1, W_ma1, a_op1, a_ma1, w_p1,
           W_op2, W_ma2, a_op2, a_ma2, w_p2,
           Ws1, Wm1, Wo1, Ws2, Wm2, Wo2,
           Wc1, Wc2, Wp1, Wp2, Vc1, bc1, Vc2, bc2, Vc3, bc3):
    f32 = jnp.float32
    a_op1 = a_op1.reshape(1, OUT)
    a_ma1 = a_ma1.reshape(1, OUT)
    a_op2 = a_op2.reshape(1, OUT)
    a_ma2 = a_ma2.reshape(1, OUT)
    w_p1 = w_p1.reshape(1, 1)
    w_p2 = w_p2.reshape(1, 1)
    bc1 = bc1.reshape(1, 64)
    bc2 = bc2.reshape(1, 64)
    bc3 = bc3.reshape(1, 1)

    proc_t = jnp.swapaxes(proc_time, 1, 2)
    adjma_t = jnp.swapaxes(ope_ma_adj, 1, 2)
    bspec = lambda shp: pl.BlockSpec(shp, lambda b: (b, 0, 0))
    full2 = lambda arr: pl.BlockSpec(arr.shape, lambda b: (0, 0))
    full3 = lambda arr: pl.BlockSpec(arr.shape, lambda b: (0, 0, 0))
    h_op2, h_ma2, misc = pl.pallas_call(
        _mega_kernel,
        grid=(B,),
        in_specs=[bspec((1, O, O)),
                  bspec((1, O, 12)), bspec((1, M, 6)),
                  full3(proc_t), bspec((1, M, O)),
                  full2(W_op1), full2(W_ma1), full2(a_op1), full2(a_ma1),
                  full2(w_p1),
                  full2(W_op2), full2(W_ma2), full2(a_op2), full2(a_ma2),
                  full2(w_p2),
                  full2(Ws1), full2(Wm1), full2(Wo1),
                  full2(Ws2), full2(Wm2), full2(Wo2)],
        out_specs=[bspec((1, O, OUT)), bspec((1, M, OUT)),
                   bspec((1, 8, 128))],
        out_shape=[jax.ShapeDtypeStruct((B, O, OUT), f32),
                   jax.ShapeDtypeStruct((B, M, OUT), f32),
                   jax.ShapeDtypeStruct((B, 8, 128), f32)],
        scratch_shapes=[pltpu.SMEM((2,), f32)],
        compiler_params=pltpu.CompilerParams(
            dimension_semantics=("arbitrary",),
            vmem_limit_bytes=112 * 1024 * 1024),
    )(ope_ope_adj, feat_opes, feat_mas, proc_t, adjma_t,
      W_op1, W_ma1, a_op1, a_ma1, w_p1,
      W_op2, W_ma2, a_op2, a_ma2, w_p2,
      Ws1, Wm1, Wo1, Ws2, Wm2, Wo2)

    head = pl.pallas_call(
        _head_kernel,
        out_shape=jax.ShapeDtypeStruct((B, 128), f32),
    )(h_op2, h_ma2, misc, Wc1, Wc2, Wp1, Wp2, Vc1, bc1, Vc2, bc2, Vc3, bc3)
    value = head[:, :1]

    return jnp.concatenate([h_op2.reshape(B, O * OUT),
                            h_ma2.reshape(B, M * OUT), value], axis=1)
